# Initial kernel scaffold; baseline (speedup 1.0000x reference)
import functools

import jax
import jax.numpy as jnp
from jax import lax
from jax.experimental import pallas as pl
from jax.experimental.pallas import tpu as pltpu
from jax.experimental.pallas import tpu_sc as plsc

B, H, W, C = 8, 256, 256, 32
N = 1000
ROI = 31
HALF = 15

_INFO = plsc.get_sparse_core_info()
NC, NS = _INFO.num_cores, _INFO.num_subcores
NW = NC * NS
ROIS_PER_W = (N + NW - 1) // NW


def _roi_body(map_hbm, rc_hbm, out_hbm, rc_v, idx_v, patch_v, sem):
    wid = lax.axis_index("s") * NC + lax.axis_index("c")

    pltpu.sync_copy(rc_hbm, rc_v)

    lane = lax.iota(jnp.int32, 16)
    c_lo = lane
    c_hi = jnp.minimum(lane + 16, ROI - 1)

    def one_roi(i, _):
        n = i * NW + wid

        @pl.when(n < N)
        def _():
            b = rc_v[3 * n]
            cy = rc_v[3 * n + 1]
            cx = rc_v[3 * n + 2]
            base = b * (H * W)
            x_lo = jnp.clip(cx - HALF + c_lo, 0, W - 1)
            x_hi = jnp.clip(cx - HALF + c_hi, 0, W - 1)

            def idx_row(r, _):
                y = jnp.clip(cy - HALF + r, 0, H - 1)
                row_base = base + y * W
                idx_v[r, pl.ds(0, 16)] = row_base + x_lo
                idx_v[r, pl.ds(16, 16)] = row_base + x_hi
                return 0

            lax.fori_loop(0, ROI, idx_row, 0)

            pltpu.async_copy(map_hbm.at[idx_v], patch_v, sem).wait()

            nt = jnp.maximum(0, HALF - cy)
            nb = jnp.maximum(0, cy - (H - 1 - HALF))
            nl = jnp.maximum(0, HALF - cx)
            nr = jnp.maximum(0, cx - (W - 1 - HALF))

            @pl.when((nt > 0) | (nb > 0) | (nl > 0) | (nr > 0))
            def _():
                zero = jnp.zeros((16,), jnp.float32)

                def zero_pixel(r, c):
                    patch_v[r, c, pl.ds(0, 16)] = zero
                    patch_v[r, c, pl.ds(16, 16)] = zero

                def zero_row(r, _):
                    def zc(c, _):
                        zero_pixel(r, c)
                        return 0
                    lax.fori_loop(0, ROI, zc, 0)
                    return 0

                lax.fori_loop(0, nt, zero_row, 0)
                lax.fori_loop(ROI - nb, ROI, zero_row, 0)

                def zero_cols(r, _):
                    def zl(c, _):
                        zero_pixel(r, c)
                        return 0
                    lax.fori_loop(0, nl, zl, 0)
                    lax.fori_loop(ROI - nr, ROI, zl, 0)
                    return 0

                lax.fori_loop(nt, ROI - nb, zero_cols, 0)

            pltpu.sync_copy(patch_v.at[:, pl.ds(0, ROI)], out_hbm.at[n])

        return 0

    lax.fori_loop(0, ROIS_PER_W, one_roi, 0)


@jax.jit
def kernel(encoded_poses, roi_centers):
    flat_map = encoded_poses.reshape(B * H * W, C)
    rc_flat = roi_centers.reshape(N * 3)

    mesh = plsc.VectorSubcoreMesh(core_axis_name="c", subcore_axis_name="s")
    run = pl.kernel(
        _roi_body,
        out_type=jax.ShapeDtypeStruct((N, ROI, ROI, C), jnp.float32),
        mesh=mesh,
        scratch_types=[
            pltpu.VMEM((N * 3,), jnp.int32),
            pltpu.VMEM((ROI, 32), jnp.int32),
            pltpu.VMEM((ROI, 32, C), jnp.float32),
            pltpu.SemaphoreType.DMA,
        ],
    )
    return run(flat_map, rc_flat)


# speedup vs baseline: 23.9226x; 23.9226x over previous
import functools

import jax
import jax.numpy as jnp
from jax import lax
from jax.experimental import pallas as pl
from jax.experimental.pallas import tpu as pltpu
from jax.experimental.pallas import tpu_sc as plsc

B, H, W, C = 8, 256, 256, 32
N = 1000
ROI = 31
HALF = 15

_INFO = plsc.get_sparse_core_info()
NC, NS = _INFO.num_cores, _INFO.num_subcores
NW = NC * NS
ROIS_PER_W = (N + NW - 1) // NW


def _roi_body(map_hbm, rc_hbm, out_hbm, rc_v, idx_v, patch_v, sem):
    wid = lax.axis_index("s") * NC + lax.axis_index("c")

    pltpu.sync_copy(rc_hbm, rc_v.at[pl.ds(0, N * 3)])

    lane = lax.iota(jnp.int32, 16)
    c_lo = lane
    c_hi = jnp.minimum(lane + 16, ROI - 1)

    def one_roi(i, _):
        n = i * NW + wid

        @pl.when(n < N)
        def _():
            vals = rc_v[pl.ds(3 * n, 16)]
            b = vals[0]
            cy = vals[1]
            cx = vals[2]
            base = b * (H * W)
            x_lo = jnp.clip(cx - HALF + c_lo, 0, W - 1)
            x_hi = jnp.clip(cx - HALF + c_hi, 0, W - 1)

            def idx_row(r, _):
                y = jnp.clip(cy - HALF + r, 0, H - 1)
                row_base = base + y * W
                idx_v[pl.ds(32 * r, 16)] = row_base + x_lo
                idx_v[pl.ds(32 * r + 16, 16)] = row_base + x_hi
                return 0

            lax.fori_loop(0, ROI, idx_row, 0)

            def fire(r, _):
                pltpu.async_copy(
                    map_hbm.at[idx_v.at[pl.ds(32 * r, ROI)]],
                    patch_v.at[r], sem)
                return 0

            def drain(r, _):
                pltpu.make_async_copy(
                    map_hbm.at[idx_v.at[pl.ds(32 * r, ROI)]],
                    patch_v.at[r], sem).wait()
                return 0

            lax.fori_loop(0, ROI, fire, 0)
            lax.fori_loop(0, ROI, drain, 0)

            nt = jnp.maximum(0, HALF - cy)
            nb = jnp.maximum(0, cy - (H - 1 - HALF))
            nl = jnp.maximum(0, HALF - cx)
            nr = jnp.maximum(0, cx - (W - 1 - HALF))

            @pl.when((nt > 0) | (nb > 0) | (nl > 0) | (nr > 0))
            def _():
                zero = jnp.zeros((16,), jnp.float32)

                def zero_pixel(r, c):
                    patch_v[r, c, pl.ds(0, 16)] = zero
                    patch_v[r, c, pl.ds(16, 16)] = zero

                def zero_row(r, _):
                    def zc(c, _):
                        zero_pixel(r, c)
                        return 0
                    lax.fori_loop(0, ROI, zc, 0)
                    return 0

                lax.fori_loop(0, nt, zero_row, 0)
                lax.fori_loop(ROI - nb, ROI, zero_row, 0)

                def zero_cols(r, _):
                    def zl(c, _):
                        zero_pixel(r, c)
                        return 0
                    lax.fori_loop(0, nl, zl, 0)
                    lax.fori_loop(ROI - nr, ROI, zl, 0)
                    return 0

                lax.fori_loop(nt, ROI - nb, zero_cols, 0)

            pltpu.sync_copy(patch_v, out_hbm.at[n])

        return 0

    lax.fori_loop(0, ROIS_PER_W, one_roi, 0)


@jax.jit
def kernel(encoded_poses, roi_centers):
    flat_map = encoded_poses.reshape(B * H * W, C)
    rc_flat = roi_centers.reshape(N * 3)

    mesh = plsc.VectorSubcoreMesh(core_axis_name="c", subcore_axis_name="s")
    run = pl.kernel(
        _roi_body,
        out_type=jax.ShapeDtypeStruct((N, ROI, ROI, C), jnp.float32),
        mesh=mesh,
        compiler_params=pltpu.CompilerParams(use_tc_tiling_on_sc=False),
        scratch_types=[
            pltpu.VMEM((N * 3 + 16,), jnp.int32),
            pltpu.VMEM((ROI * 32,), jnp.int32),
            pltpu.VMEM((ROI, ROI, C), jnp.float32),
            pltpu.SemaphoreType.DMA,
        ],
    )
    return run(flat_map, rc_flat)
